# baseline (speedup 1.0000x reference)
import functools

import jax
import jax.numpy as jnp
from jax.experimental import pallas as pl
from jax.experimental.pallas import tpu as pltpu

_SLOPE = 0.05


def _se_kernel(x_ref, w1_ref, b1_ref, w2_ref, b2_ref, o_ref, *, inv_hw):
    x = x_ref[...]
    pooled = jnp.sum(x, axis=1) * inv_hw
    dims = (((1,), (1,)), ((), ()))
    h = jax.lax.dot_general(pooled, w1_ref[...], dims,
                            preferred_element_type=jnp.float32) + b1_ref[...]
    h = jnp.maximum(h, 0.0) + _SLOPE * jnp.minimum(h, 0.0)
    g = jax.lax.dot_general(h, w2_ref[...], dims,
                            preferred_element_type=jnp.float32) + b2_ref[...]
    s = jax.nn.sigmoid(g)
    o_ref[...] = x * s[:, None, :]


def kernel(x_nchw, w1, b1, w2, b2):
    B, C, H, W = x_nchw.shape
    HW = H * W
    xv = x_nchw.transpose(0, 2, 3, 1).reshape(B, HW, C)

    slab_bytes = pl.cdiv(HW, 8) * 8 * pl.cdiv(C, 128) * 128 * xv.dtype.itemsize
    group = max(1, min(B, (13 << 20) // (4 * slab_bytes)))
    while B % group:
        group -= 1
    n_steps = B // group

    vmem = 4 * group * slab_bytes + 2 * C * C * 4 + (4 << 20)

    out = pl.pallas_call(
        functools.partial(_se_kernel, inv_hw=1.0 / HW),
        out_shape=jax.ShapeDtypeStruct((B, HW, C), xv.dtype),
        grid=(n_steps,),
        in_specs=[
            pl.BlockSpec((group, HW, C), lambda i: (i, 0, 0)),
            pl.BlockSpec((C, C), lambda i: (0, 0)),
            pl.BlockSpec((1, C), lambda i: (0, 0)),
            pl.BlockSpec((C, C), lambda i: (0, 0)),
            pl.BlockSpec((1, C), lambda i: (0, 0)),
        ],
        out_specs=pl.BlockSpec((group, HW, C), lambda i: (i, 0, 0)),
        compiler_params=pltpu.CompilerParams(
            dimension_semantics=("arbitrary",),
            vmem_limit_bytes=int(min(vmem, 60 << 20))),
    )(xv, w1, b1.reshape(1, C), w2, b2.reshape(1, C))

    return out.reshape(B, H, W, C).transpose(0, 3, 1, 2)


# speedup vs baseline: 1.1876x; 1.1876x over previous
import functools

import jax
import jax.numpy as jnp
from jax.experimental import pallas as pl
from jax.experimental.pallas import tpu as pltpu

_SLOPE = 0.05


def _se_kernel(x_ref, w1_ref, b1_ref, w2_ref, b2_ref, o_ref, *, inv_hw):
    x = x_ref[...]
    pooled = jnp.sum(x, axis=1) * inv_hw
    dims = (((1,), (1,)), ((), ()))
    h = jax.lax.dot_general(pooled, w1_ref[...], dims,
                            preferred_element_type=jnp.float32) + b1_ref[...]
    h = jnp.maximum(h, 0.0) + _SLOPE * jnp.minimum(h, 0.0)
    g = jax.lax.dot_general(h, w2_ref[...], dims,
                            preferred_element_type=jnp.float32) + b2_ref[...]
    s = jax.nn.sigmoid(g)
    o_ref[...] = x * s[:, None, :]


def kernel(x_nchw, w1, b1, w2, b2):
    B, C, H, W = x_nchw.shape
    HW = H * W
    xv = x_nchw.transpose(0, 2, 3, 1).reshape(B, HW, C)

    slab_bytes = pl.cdiv(HW, 8) * 8 * pl.cdiv(C, 128) * 128 * xv.dtype.itemsize
    group = max(1, min(B, (52 << 20) // (4 * slab_bytes)))
    while B % group:
        group -= 1
    n_steps = B // group

    vmem = 4 * group * slab_bytes + 2 * C * C * 4 + (4 << 20)

    out = pl.pallas_call(
        functools.partial(_se_kernel, inv_hw=1.0 / HW),
        out_shape=jax.ShapeDtypeStruct((B, HW, C), xv.dtype),
        grid=(n_steps,),
        in_specs=[
            pl.BlockSpec((group, HW, C), lambda i: (i, 0, 0)),
            pl.BlockSpec((C, C), lambda i: (0, 0)),
            pl.BlockSpec((1, C), lambda i: (0, 0)),
            pl.BlockSpec((C, C), lambda i: (0, 0)),
            pl.BlockSpec((1, C), lambda i: (0, 0)),
        ],
        out_specs=pl.BlockSpec((group, HW, C), lambda i: (i, 0, 0)),
        compiler_params=pltpu.CompilerParams(
            dimension_semantics=("arbitrary",),
            vmem_limit_bytes=int(min(vmem, 60 << 20))),
    )(xv, w1, b1.reshape(1, C), w2, b2.reshape(1, C))

    return out.reshape(B, H, W, C).transpose(0, 3, 1, 2)
